# baseline (speedup 1.0000x reference)
import jax
import jax.numpy as jnp
from jax.experimental import pallas as pl

B, NK, NF, D = 4096, 8000, 1024, 64
BB = 256


def _fused_kernel(prot_ref, comp_ref, Wp_ref, bp_ref, Wc_ref, bc_ref,
                  Wb_ref, bb_ref, Wl_ref, bl_ref, out_ref):
    p = jnp.dot(prot_ref[...], Wp_ref[...], preferred_element_type=jnp.float32)
    p = jnp.maximum(p + bp_ref[...], 0.0)
    c = jnp.dot(comp_ref[...], Wc_ref[...], preferred_element_type=jnp.float32)
    c = jnp.maximum(c + bc_ref[...], 0.0)
    M = (p[:, :, None] * c[:, None, :]).reshape(BB, D * D)
    joined = jax.lax.dot_general(M, Wb_ref[...], (((1,), (1,)), ((), ())),
                                 preferred_element_type=jnp.float32)
    joined = jnp.maximum(joined + bb_ref[...], 0.0)
    out_ref[...] = jnp.dot(joined, Wl_ref[...],
                           preferred_element_type=jnp.float32) + bl_ref[...]


def kernel(protein_input, compound_input, Wp, bp, Wc, bc, Wb, bb, Wl, bl):
    Wb2 = Wb.reshape(D, D * D)
    grid = (B // BB,)
    out = pl.pallas_call(
        _fused_kernel,
        grid=grid,
        in_specs=[
            pl.BlockSpec((BB, NK), lambda i: (i, 0)),
            pl.BlockSpec((BB, NF), lambda i: (i, 0)),
            pl.BlockSpec((NK, D), lambda i: (0, 0)),
            pl.BlockSpec((1, D), lambda i: (0, 0)),
            pl.BlockSpec((NF, D), lambda i: (0, 0)),
            pl.BlockSpec((1, D), lambda i: (0, 0)),
            pl.BlockSpec((D, D * D), lambda i: (0, 0)),
            pl.BlockSpec((1, D), lambda i: (0, 0)),
            pl.BlockSpec((D, 1), lambda i: (0, 0)),
            pl.BlockSpec((1, 1), lambda i: (0, 0)),
        ],
        out_specs=pl.BlockSpec((BB, 1), lambda i: (i, 0)),
        out_shape=jax.ShapeDtypeStruct((B, 1), jnp.float32),
    )(protein_input, compound_input, Wp, bp.reshape(1, D), Wc,
      bc.reshape(1, D), Wb2, bb.reshape(1, D), Wl, bl.reshape(1, 1))
    return out


# speedup vs baseline: 1.1415x; 1.1415x over previous
import jax
import jax.numpy as jnp
import numpy as np
from jax.experimental import pallas as pl

B, NK, NF, D = 4096, 8000, 1024, 64
BB = 128


def _fused_kernel(prot_ref, comp_ref, Wp_ref, bp_ref, Wc_ref, bc_ref,
                  Wb_ref, bb_ref, Wl_ref, bl_ref, S_ref, out_ref):
    p = jnp.dot(prot_ref[...], Wp_ref[...], preferred_element_type=jnp.float32)
    p = jnp.maximum(p + bp_ref[...], 0.0)
    c = jnp.dot(comp_ref[...], Wc_ref[...], preferred_element_type=jnp.float32)
    c = jnp.maximum(c + bc_ref[...], 0.0)
    u = jnp.dot(c, Wb_ref[...], preferred_element_type=jnp.float32)
    Z = u * jnp.tile(p, (1, D))
    joined = jnp.dot(Z, S_ref[...], preferred_element_type=jnp.float32)
    joined = jnp.maximum(joined + bb_ref[...], 0.0)
    out_ref[...] = jnp.dot(joined, Wl_ref[...],
                           preferred_element_type=jnp.float32) + bl_ref[...]


def kernel(protein_input, compound_input, Wp, bp, Wc, bc, Wb, bb, Wl, bl):
    Wb2 = jnp.transpose(Wb, (2, 0, 1)).reshape(D, D * D)
    S = jnp.asarray(np.kron(np.eye(D, dtype=np.float32),
                            np.ones((D, 1), dtype=np.float32)))
    grid = (B // BB,)
    out = pl.pallas_call(
        _fused_kernel,
        grid=grid,
        in_specs=[
            pl.BlockSpec((BB, NK), lambda i: (i, 0)),
            pl.BlockSpec((BB, NF), lambda i: (i, 0)),
            pl.BlockSpec((NK, D), lambda i: (0, 0)),
            pl.BlockSpec((1, D), lambda i: (0, 0)),
            pl.BlockSpec((NF, D), lambda i: (0, 0)),
            pl.BlockSpec((1, D), lambda i: (0, 0)),
            pl.BlockSpec((D, D * D), lambda i: (0, 0)),
            pl.BlockSpec((1, D), lambda i: (0, 0)),
            pl.BlockSpec((D, 1), lambda i: (0, 0)),
            pl.BlockSpec((1, 1), lambda i: (0, 0)),
            pl.BlockSpec((D * D, D), lambda i: (0, 0)),
        ],
        out_specs=pl.BlockSpec((BB, 1), lambda i: (i, 0)),
        out_shape=jax.ShapeDtypeStruct((B, 1), jnp.float32),
    )(protein_input, compound_input, Wp, bp.reshape(1, D), Wc,
      bc.reshape(1, D), Wb2, bb.reshape(1, D), Wl, bl.reshape(1, 1), S)
    return out
